# baseline (speedup 1.0000x reference)
import functools

import jax
import jax.numpy as jnp
from jax import lax
from jax.experimental import pallas as pl
from jax.experimental.pallas import tpu as pltpu
from jax.experimental.pallas import tpu_sc as plsc

_NUM_WORKERS = 32
_CHUNK = 64
_RING = 2


def kernel(x, pe_table):
    b, size, dx = x.shape
    dim = pe_table.shape[-1]
    rows = size // _NUM_WORKERS
    nchunks = rows // _CHUNK
    total = b * nchunks
    mesh = plsc.VectorSubcoreMesh(core_axis_name="c", subcore_axis_name="s")

    @functools.partial(
        pl.kernel,
        mesh=mesh,
        out_type=jax.ShapeDtypeStruct((b, size, dx + dim), x.dtype),
        scratch_types=[
            pltpu.MemorySpace.VMEM((_RING, _CHUNK, dx + dim), x.dtype),
            pltpu.SemaphoreType.DMA((_RING,)),
            pltpu.SemaphoreType.DMA((_RING,)),
            pltpu.SemaphoreType.DMA((_RING,)),
        ],
    )
    def run(x_hbm, pe_hbm, out_hbm, buf, sem_xl, sem_pl, sem_st):
        wid = lax.axis_index("s") * 2 + lax.axis_index("c")
        s0 = wid * rows

        x_loads, pe_loads, stores = [], [], []
        for i in range(total):
            bb, r0 = i // nchunks, (i % nchunks) * _CHUNK
            slot = i % _RING
            x_loads.append(
                pltpu.make_async_copy(
                    x_hbm.at[bb, pl.ds(s0 + r0, _CHUNK), :],
                    buf.at[slot, :, pl.ds(0, dx)],
                    sem_xl.at[slot],
                )
            )
            pe_loads.append(
                pltpu.make_async_copy(
                    pe_hbm.at[pl.ds(s0 + r0, _CHUNK), :],
                    buf.at[slot, :, pl.ds(dx, dim)],
                    sem_pl.at[slot],
                )
            )
            stores.append(
                pltpu.make_async_copy(
                    buf.at[slot],
                    out_hbm.at[bb, pl.ds(s0 + r0, _CHUNK), :],
                    sem_st.at[slot],
                )
            )

        for i in range(min(_RING, total)):
            x_loads[i].start()
            pe_loads[i].start()
        for i in range(total):
            x_loads[i].wait()
            pe_loads[i].wait()
            stores[i].start()
            if i + _RING < total:
                stores[i].wait()
                x_loads[i + _RING].start()
                pe_loads[i + _RING].start()
        for i in range(max(0, total - _RING), total):
            stores[i].wait()

    return run(x, pe_table)


# speedup vs baseline: 1.2076x; 1.2076x over previous
import functools

import jax
import jax.numpy as jnp
from jax import lax
from jax.experimental import pallas as pl
from jax.experimental.pallas import tpu as pltpu
from jax.experimental.pallas import tpu_sc as plsc

_NUM_WORKERS = 32


def kernel(x, pe_table):
    b, size, dx = x.shape
    dim = pe_table.shape[-1]
    rows = size // _NUM_WORKERS
    mesh = plsc.VectorSubcoreMesh(core_axis_name="c", subcore_axis_name="s")

    @functools.partial(
        pl.kernel,
        mesh=mesh,
        out_type=jax.ShapeDtypeStruct((b, size, dx + dim), x.dtype),
        scratch_types=[
            pltpu.MemorySpace.VMEM((rows, dim), x.dtype),
            pltpu.MemorySpace.VMEM((2, rows, dx), x.dtype),
            pltpu.SemaphoreType.DMA,
            pltpu.SemaphoreType.DMA((2,)),
            pltpu.SemaphoreType.DMA((2,)),
            pltpu.SemaphoreType.DMA,
        ],
    )
    def run(x_hbm, pe_hbm, out_hbm, pebuf, xbuf, sem_pe, sem_xl, sem_xs, sem_ps):
        wid = lax.axis_index("s") * 2 + lax.axis_index("c")
        s0 = wid * rows
        pe_load = pltpu.make_async_copy(pe_hbm.at[pl.ds(s0, rows), :], pebuf, sem_pe)
        pe_load.start()
        x_loads = [
            pltpu.make_async_copy(
                x_hbm.at[bb, pl.ds(s0, rows), :], xbuf.at[bb % 2], sem_xl.at[bb % 2]
            )
            for bb in range(b)
        ]
        x_stores = [
            pltpu.make_async_copy(
                xbuf.at[bb % 2],
                out_hbm.at[bb, pl.ds(s0, rows), pl.ds(0, dx)],
                sem_xs.at[bb % 2],
            )
            for bb in range(b)
        ]
        pe_stores = [
            pltpu.make_async_copy(
                pebuf, out_hbm.at[bb, pl.ds(s0, rows), pl.ds(dx, dim)], sem_ps
            )
            for bb in range(b)
        ]
        x_loads[0].start()
        if b > 1:
            x_loads[1].start()
        pe_load.wait()
        for st in pe_stores:
            st.start()
        for bb in range(b):
            x_loads[bb].wait()
            x_stores[bb].start()
            if bb + 2 < b:
                x_stores[bb].wait()
                x_loads[bb + 2].start()
        for bb in range(max(0, b - 2), b):
            x_stores[bb].wait()
        for bb in range(b):
            pe_stores[bb].wait()

    return run(x, pe_table)


# speedup vs baseline: 1.2130x; 1.0045x over previous
import functools

import jax
from jax import lax
from jax.experimental import pallas as pl
from jax.experimental.pallas import tpu as pltpu
from jax.experimental.pallas import tpu_sc as plsc

_NUM_WORKERS = 32


def kernel(x, pe_table):
    b, size, dx = x.shape
    dim = pe_table.shape[-1]
    rows = size // _NUM_WORKERS
    mesh = plsc.VectorSubcoreMesh(core_axis_name="c", subcore_axis_name="s")

    @functools.partial(
        pl.kernel,
        mesh=mesh,
        out_type=jax.ShapeDtypeStruct((b, size, dx + dim), x.dtype),
        scratch_types=[
            pltpu.MemorySpace.VMEM((rows, dim), x.dtype),
            pltpu.MemorySpace.VMEM((2, rows, dx), x.dtype),
            pltpu.SemaphoreType.DMA,
            pltpu.SemaphoreType.DMA((2,)),
            pltpu.SemaphoreType.DMA((2,)),
            pltpu.SemaphoreType.DMA,
        ],
    )
    def run(x_hbm, pe_hbm, out_hbm, pebuf, xbuf, sem_pe, sem_xl, sem_xs, sem_ps):
        wid = lax.axis_index("s") * 2 + lax.axis_index("c")
        s0 = wid * rows
        pe_load = pltpu.make_async_copy(pe_hbm.at[pl.ds(s0, rows), :], pebuf, sem_pe)
        pe_load.start()
        x_loads = [
            pltpu.make_async_copy(
                x_hbm.at[bb, pl.ds(s0, rows), :], xbuf.at[bb % 2], sem_xl.at[bb % 2]
            )
            for bb in range(b)
        ]
        x_stores = [
            pltpu.make_async_copy(
                xbuf.at[bb % 2],
                out_hbm.at[bb, pl.ds(s0, rows), pl.ds(0, dx)],
                sem_xs.at[bb % 2],
            )
            for bb in range(b)
        ]
        pe_stores = [
            pltpu.make_async_copy(
                pebuf, out_hbm.at[bb, pl.ds(s0, rows), pl.ds(dx, dim)], sem_ps
            )
            for bb in range(b)
        ]
        x_loads[0].start()
        if b > 1:
            x_loads[1].start()
        pe_load.wait()
        for st in pe_stores:
            st.start()
        for bb in range(b):
            x_loads[bb].wait()
            x_stores[bb].start()
            if bb + 2 < b:
                x_stores[bb].wait()
                x_loads[bb + 2].start()
        for bb in range(max(0, b - 2), b):
            x_stores[bb].wait()
        for bb in range(b):
            pe_stores[bb].wait()

    return run(x, pe_table)
